# baseline (speedup 1.0000x reference)
import functools

import jax
import jax.numpy as jnp
from jax import lax
from jax.experimental import pallas as pl
from jax.experimental.pallas import tpu as pltpu
from jax.experimental.pallas import tpu_sc as plsc

_B, _S = 4, 2048
_HIDDEN = 768
_NUM_POS = 8128
_N = _B * _S

_NC, _NS = 2, 16
_NW = _NC * _NS
_TPW = _N // _NW
_C = 4
_NCHUNK = _TPW // _C
_NSTREAM = 10
_NPLANE = 11
_LANES = 16
_NVH = _HIDDEN // _LANES


def _sc_gather_sum(idx_all, tok_flat, w_word, w_x, w_y, w_center, w_dist, w_angle,
                   w_tok, w_pos):
    mesh = plsc.VectorSubcoreMesh(core_axis_name="c", subcore_axis_name="s")

    @functools.partial(
        pl.kernel,
        out_type=jax.ShapeDtypeStruct((_N, _HIDDEN), jnp.float32),
        mesh=mesh,
        scratch_types=[
            pltpu.VMEM((_NSTREAM, _TPW), jnp.int32),
            pltpu.VMEM((_TPW * _LANES,), jnp.float32),
            pltpu.VMEM((_NPLANE, _C, _HIDDEN), jnp.float32),
            pltpu.VMEM((_NPLANE, _C, _HIDDEN), jnp.float32),
            pltpu.VMEM((_C, _HIDDEN), jnp.float32),
            pltpu.VMEM((2, _HIDDEN), jnp.float32),
            pltpu.SemaphoreType.DMA,
            pltpu.SemaphoreType.DMA,
            pltpu.SemaphoreType.DMA,
        ],
    )
    def kern(idx_hbm, tok_hbm, t_word, t_x, t_y, t_c, t_d, t_a, t_t, t_p, out_hbm,
             idx_v, tok_ids, g0, g1, ostg, tokv, semg0, semg1, semo):
        wid = lax.axis_index("s") * _NC + lax.axis_index("c")
        base = wid * _TPW
        s_base = base % _S

        pltpu.sync_copy(idx_hbm.at[:, pl.ds(base, _TPW)], idx_v)
        pltpu.sync_copy(t_t, tokv)
        pltpu.sync_copy(tok_hbm.at[pl.ds(base * _LANES, _TPW * _LANES)], tok_ids)

        tables = (t_word, t_x, t_x, t_y, t_y, t_c, t_c, t_d, t_d, t_a)

        def issue(ci, gb, semg):
            pltpu.async_copy(t_p.at[pl.ds(s_base + ci * _C, _C)],
                             gb.at[_NSTREAM], semg)
            for j in range(_NSTREAM):
                pltpu.async_copy(tables[j].at[idx_v.at[j, pl.ds(ci * _C, _C)]],
                                 gb.at[j], semg)

        def drain(gb, semg):
            for _ in range(_NPLANE):
                pltpu.make_async_copy(t_p.at[pl.ds(0, _C)], gb.at[0],
                                      semg).wait()

        def drain_out():
            pltpu.make_async_copy(t_p.at[pl.ds(0, _C)], ostg, semo).wait()

        def accumulate(ci, gb):
            for t in range(_C):
                @plsc.parallel_loop(0, _NVH, unroll=1)
                def _(j):
                    sl = pl.ds(j * _LANES, _LANES)
                    tok_sel = tok_ids[pl.ds((ci * _C + t) * _LANES,
                                            _LANES)] > 0.5
                    v = gb[0, t, sl]
                    for k in range(1, _NPLANE):
                        v = v + gb[k, t, sl]
                    r0 = tokv[0, sl]
                    r1 = tokv[1, sl]
                    v = v + jnp.where(tok_sel, r1, r0)
                    ostg[t, sl] = v

        def half(ci, gb, semg, gb_nxt, semg_nxt, first):
            @pl.when(ci + 1 < _NCHUNK)
            def _():
                issue(ci + 1, gb_nxt, semg_nxt)
            drain(gb, semg)
            if first:
                @pl.when(ci > 0)
                def _():
                    drain_out()
            else:
                drain_out()
            accumulate(ci, gb)
            pltpu.async_copy(ostg, out_hbm.at[pl.ds(base + ci * _C, _C)],
                             semo)

        issue(0, g0, semg0)

        def pair(cp, _):
            ci = cp * 2
            half(ci, g0, semg0, g1, semg1, True)
            half(ci + 1, g1, semg1, g0, semg0, False)
            return 0

        lax.fori_loop(0, _NCHUNK // 2, pair, 0)
        drain_out()

    return kern(idx_all, tok_flat, w_word, w_x, w_y, w_center, w_dist, w_angle,
                w_tok, w_pos)


def kernel(input_ids, bbox, token_type_ids, W_word, W_pos, W_x, W_y,
           W_center, W_dist, W_angle, W_tok):
    bbox = bbox.astype(jnp.int32)
    b0, b1, b2, b3 = bbox[..., 0], bbox[..., 1], bbox[..., 2], bbox[..., 3]
    cx = jnp.clip((b0 + b2) // 2, 0, _NUM_POS - 1)
    cy = jnp.clip((b1 + b3) // 2, 0, _NUM_POS - 1)
    w = jnp.clip(jnp.abs(b2 - b0), 0, _NUM_POS - 1)
    h = jnp.clip(jnp.abs(b3 - b1), 0, _NUM_POS - 1)
    ang = jnp.arctan2(h.astype(jnp.float32) + 1e-6, w.astype(jnp.float32) + 1e-6)
    ang_idx = jnp.clip((ang / (jnp.pi / 2.0) * (_NUM_POS - 1)).astype(jnp.int32),
                       0, _NUM_POS - 1)
    idx_all = jnp.stack([
        input_ids.reshape(-1).astype(jnp.int32),
        b0.reshape(-1), b2.reshape(-1),
        b1.reshape(-1), b3.reshape(-1),
        cx.reshape(-1), cy.reshape(-1),
        w.reshape(-1), h.reshape(-1),
        ang_idx.reshape(-1),
    ])
    tok_flat = jnp.repeat(token_type_ids.reshape(-1).astype(jnp.float32), _LANES)
    out = _sc_gather_sum(idx_all, tok_flat, W_word, W_x, W_y, W_center,
                         W_dist, W_angle, W_tok, W_pos)
    return out.reshape(_B, _S, _HIDDEN)


# speedup vs baseline: 1.0141x; 1.0141x over previous
import functools

import jax
import jax.numpy as jnp
from jax import lax
from jax.experimental import pallas as pl
from jax.experimental.pallas import tpu as pltpu
from jax.experimental.pallas import tpu_sc as plsc

_B, _S = 4, 2048
_HIDDEN = 768
_NUM_POS = 8128
_N = _B * _S

_NC, _NS = 2, 16
_NW = _NC * _NS
_TPW = _N // _NW
_C = 4
_NCHUNK = _TPW // _C
_NSTREAM = 10
_NPLANE = 11
_LANES = 16
_NVH = _HIDDEN // _LANES


def _sc_gather_sum(idx_all, tok_flat, w_word, w_x, w_y, w_center, w_dist, w_angle,
                   w_tok, w_pos):
    mesh = plsc.VectorSubcoreMesh(core_axis_name="c", subcore_axis_name="s")

    @functools.partial(
        pl.kernel,
        out_type=jax.ShapeDtypeStruct((_N, _HIDDEN), jnp.float32),
        mesh=mesh,
        scratch_types=[
            pltpu.VMEM((_NSTREAM, _TPW), jnp.int32),
            pltpu.VMEM((_TPW * _LANES,), jnp.float32),
            pltpu.VMEM((_NPLANE, _C, _HIDDEN), jnp.float32),
            pltpu.VMEM((_NPLANE, _C, _HIDDEN), jnp.float32),
            pltpu.VMEM((_C, _HIDDEN), jnp.float32),
            pltpu.VMEM((2, _HIDDEN), jnp.float32),
            pltpu.SemaphoreType.DMA,
            pltpu.SemaphoreType.DMA,
            pltpu.SemaphoreType.DMA,
        ],
    )
    def kern(idx_hbm, tok_hbm, t_word, t_x, t_y, t_c, t_d, t_a, t_t, t_p, out_hbm,
             idx_v, tok_ids, g0, g1, ostg, tokv, semg0, semg1, semo):
        wid = lax.axis_index("s") * _NC + lax.axis_index("c")
        base = wid * _TPW
        s_base = base % _S

        pltpu.sync_copy(idx_hbm.at[:, pl.ds(base, _TPW)], idx_v)
        pltpu.sync_copy(t_t, tokv)
        pltpu.sync_copy(tok_hbm.at[pl.ds(base * _LANES, _TPW * _LANES)], tok_ids)

        tables = (t_word, t_x, t_x, t_y, t_y, t_c, t_c, t_d, t_d, t_a)

        def issue(ci, gb, semg):
            pltpu.async_copy(t_p.at[pl.ds(s_base + ci * _C, _C)],
                             gb.at[_NSTREAM], semg)
            for j in range(_NSTREAM):
                pltpu.async_copy(tables[j].at[idx_v.at[j, pl.ds(ci * _C, _C)]],
                                 gb.at[j], semg)

        def drain(gb, semg):
            for _ in range(_NPLANE):
                pltpu.make_async_copy(t_p.at[pl.ds(0, _C)], gb.at[0],
                                      semg).wait()

        def drain_out():
            pltpu.make_async_copy(t_p.at[pl.ds(0, _C)], ostg, semo).wait()

        def accumulate(ci, gb):
            for t in range(_C):
                @plsc.parallel_loop(0, _NVH, unroll=2)
                def _(j):
                    sl = pl.ds(j * _LANES, _LANES)
                    tok_sel = tok_ids[pl.ds((ci * _C + t) * _LANES,
                                            _LANES)] > 0.5
                    v = gb[0, t, sl]
                    for k in range(1, _NPLANE):
                        v = v + gb[k, t, sl]
                    r0 = tokv[0, sl]
                    r1 = tokv[1, sl]
                    v = v + jnp.where(tok_sel, r1, r0)
                    ostg[t, sl] = v

        def half(ci, gb, semg, gb_nxt, semg_nxt, first):
            @pl.when(ci + 1 < _NCHUNK)
            def _():
                issue(ci + 1, gb_nxt, semg_nxt)
            drain(gb, semg)
            if first:
                @pl.when(ci > 0)
                def _():
                    drain_out()
            else:
                drain_out()
            accumulate(ci, gb)
            pltpu.async_copy(ostg, out_hbm.at[pl.ds(base + ci * _C, _C)],
                             semo)

        issue(0, g0, semg0)

        def pair(cp, _):
            ci = cp * 2
            half(ci, g0, semg0, g1, semg1, True)
            half(ci + 1, g1, semg1, g0, semg0, False)
            return 0

        lax.fori_loop(0, _NCHUNK // 2, pair, 0)
        drain_out()

    return kern(idx_all, tok_flat, w_word, w_x, w_y, w_center, w_dist, w_angle,
                w_tok, w_pos)


def kernel(input_ids, bbox, token_type_ids, W_word, W_pos, W_x, W_y,
           W_center, W_dist, W_angle, W_tok):
    bbox = bbox.astype(jnp.int32)
    b0, b1, b2, b3 = bbox[..., 0], bbox[..., 1], bbox[..., 2], bbox[..., 3]
    cx = jnp.clip((b0 + b2) // 2, 0, _NUM_POS - 1)
    cy = jnp.clip((b1 + b3) // 2, 0, _NUM_POS - 1)
    w = jnp.clip(jnp.abs(b2 - b0), 0, _NUM_POS - 1)
    h = jnp.clip(jnp.abs(b3 - b1), 0, _NUM_POS - 1)
    ang = jnp.arctan2(h.astype(jnp.float32) + 1e-6, w.astype(jnp.float32) + 1e-6)
    ang_idx = jnp.clip((ang / (jnp.pi / 2.0) * (_NUM_POS - 1)).astype(jnp.int32),
                       0, _NUM_POS - 1)
    idx_all = jnp.stack([
        input_ids.reshape(-1).astype(jnp.int32),
        b0.reshape(-1), b2.reshape(-1),
        b1.reshape(-1), b3.reshape(-1),
        cx.reshape(-1), cy.reshape(-1),
        w.reshape(-1), h.reshape(-1),
        ang_idx.reshape(-1),
    ])
    tok_flat = jnp.repeat(token_type_ids.reshape(-1).astype(jnp.float32), _LANES)
    out = _sc_gather_sum(idx_all, tok_flat, W_word, W_x, W_y, W_center,
                         W_dist, W_angle, W_tok, W_pos)
    return out.reshape(_B, _S, _HIDDEN)
